# baseline (speedup 1.0000x reference)
import functools

import jax
import jax.numpy as jnp
from jax import lax
from jax.experimental import pallas as pl
from jax.experimental.pallas import tpu as pltpu
from jax.experimental.pallas import tpu_sc as plsc

NUM_ROWS = 100000
NUM_COLS = 128
NUM_SEG = 512
NC = 2
NS = 16
ROWS_PER_CORE = NUM_ROWS // NC
BLK = 120
FULL_BLKS = ROWS_PER_CORE // BLK
BLKS_PER_TILE = FULL_BLKS // NS
REM = ROWS_PER_CORE - FULL_BLKS * BLK
SEG_PER_TILE = NUM_SEG // NS
NBUF = 2
GROUPS = BLKS_PER_TILE // NBUF
NCG = NUM_COLS // 16
ROW_UNROLL = 8


def _run_reduce_rows(xb, ib, nrows, carry, acc_l):
    iota16 = lax.iota(jnp.int32, 16)

    def row_step(r, carry):
        curb, accs = carry
        idb = plsc.load_gather(ib, [jnp.zeros((16,), jnp.int32) + r])
        fl = idb != curb
        for j in range(NCG):
            plsc.addupdate_scatter(
                acc_l, [curb, iota16 + 16 * j], accs[j], mask=fl)
        new_accs = []
        for j in range(NCG):
            xr = xb[r, pl.ds(j * 16, 16)]
            new_accs.append(jnp.where(fl, xr, accs[j] + xr))
        return idb, new_accs

    def group(u, carry):
        for v in range(ROW_UNROLL):
            carry = row_step(u * ROW_UNROLL + v, carry)
        return carry

    return lax.fori_loop(0, nrows // ROW_UNROLL, group, carry)


def _seg_sum_body(x_hbm, ids_hbm, part_hbm, scratch):
    (idxs, xbs, zbuf, acc_l, acc_sh, ibuf, sld) = scratch
    c = lax.axis_index("c")
    s = lax.axis_index("s")

    base = c * ROWS_PER_CORE + s * (BLKS_PER_TILE * BLK)

    def start(b, k):
        row0 = pl.multiple_of(base + b * BLK, 8)
        pltpu.async_copy(ids_hbm.at[pl.ds(row0, BLK)],
                         idxs[k].at[pl.ds(0, BLK)], sld[k])
        pltpu.async_copy(x_hbm.at[pl.ds(row0, BLK)], xbs[k], sld[k])

    def wait(k):
        pltpu.make_async_copy(ids_hbm.at[pl.ds(0, BLK)],
                              idxs[k].at[pl.ds(0, BLK)], sld[k]).wait()
        pltpu.make_async_copy(x_hbm.at[pl.ds(0, BLK)], xbs[k], sld[k]).wait()

    for k in range(NBUF):
        start(k, k)

    def zrow(i, carry):
        for j in range(NCG):
            zbuf[i, pl.ds(j * 16, 16)] = jnp.zeros((16,), jnp.float32)
        return carry

    lax.fori_loop(0, SEG_PER_TILE, zrow, 0)
    pltpu.sync_copy(zbuf, acc_sh.at[pl.ds(s * SEG_PER_TILE, SEG_PER_TILE)])

    def zrow_l(i, carry):
        for j in range(NCG):
            acc_l[i, pl.ds(j * 16, 16)] = jnp.zeros((16,), jnp.float32)
        return carry

    lax.fori_loop(0, NUM_SEG + 1, zrow_l, 0)

    for q in range(NUM_SEG // 128):
        for t in range(128 // 16):
            ibuf[q, pl.ds(t * 16, 16)] = (
                lax.iota(jnp.int32, 16) + (q * 128 + t * 16))

    plsc.subcore_barrier()

    zero_accs = [jnp.zeros((16,), jnp.float32) for _ in range(NCG)]
    sentinel = jnp.full((16,), NUM_SEG, jnp.int32)
    carry = (sentinel, zero_accs)

    def grp(g, carry):
        for k in range(NBUF):
            b = NBUF * g + k
            wait(k)
            carry = _run_reduce_rows(xbs[k], idxs[k], BLK, carry, acc_l)

            @pl.when(g < GROUPS - 1)
            def _():
                start(b + NBUF, k)
        return carry

    carry = lax.fori_loop(0, GROUPS, grp, carry)

    def rem_case(carry):
        row0 = pl.multiple_of(c * ROWS_PER_CORE + FULL_BLKS * BLK, 8)
        pltpu.sync_copy(ids_hbm.at[pl.ds(row0, REM)],
                        idxs[0].at[pl.ds(0, REM)])
        pltpu.sync_copy(x_hbm.at[pl.ds(row0, REM)], xbs[0].at[pl.ds(0, REM)])
        return _run_reduce_rows(xbs[0], idxs[0], REM, carry, acc_l)

    carry = lax.cond(s == NS - 1, rem_case, lambda car: car, carry)

    curb, accs = carry
    iota16 = lax.iota(jnp.int32, 16)
    for j in range(NCG):
        plsc.addupdate_scatter(acc_l, [curb, iota16 + 16 * j], accs[j])

    for q in range(NUM_SEG // 128):
        pltpu.sync_copy(acc_l.at[pl.ds(q * 128, 128)],
                        acc_sh.at[ibuf.at[q]], add=True)

    plsc.subcore_barrier()

    @pl.when(s == 0)
    def _():
        pltpu.sync_copy(acc_sh, part_hbm.at[c])


def _body_wrapper(x_hbm, ids_hbm, part_hbm,
                  i0, i1, x0, x1, zbuf, acc_l, acc_sh, ibuf, l0, l1):
    _seg_sum_body(x_hbm, ids_hbm, part_hbm,
                  ((i0, i1), (x0, x1), zbuf, acc_l, acc_sh, ibuf, (l0, l1)))


def _combine_body(p_ref, o_ref):
    o_ref[...] = p_ref[0] + p_ref[1]


@jax.jit
def _seg_sum(x, ids):
    mesh = plsc.VectorSubcoreMesh(core_axis_name="c", subcore_axis_name="s")
    parts = functools.partial(
        pl.kernel,
        out_type=jax.ShapeDtypeStruct((NC, NUM_SEG, NUM_COLS), jnp.float32),
        mesh=mesh,
        compiler_params=pltpu.CompilerParams(needs_layout_passes=False),
        scratch_types=(
            [pltpu.VMEM((BLK + 8,), jnp.int32)] * 2
            + [pltpu.VMEM((BLK, NUM_COLS), jnp.float32)] * 2
            + [pltpu.VMEM((SEG_PER_TILE, NUM_COLS), jnp.float32)]
            + [pltpu.VMEM((NUM_SEG + 1, NUM_COLS), jnp.float32)]
            + [pltpu.VMEM_SHARED((NUM_SEG, NUM_COLS), jnp.float32)]
            + [pltpu.VMEM((NUM_SEG // 128, 128), jnp.int32)]
            + [pltpu.SemaphoreType.DMA] * 2
        ),
    )(_body_wrapper)(x, ids)
    return pl.pallas_call(
        _combine_body,
        out_shape=jax.ShapeDtypeStruct((NUM_SEG, NUM_COLS), jnp.float32),
    )(parts)


def kernel(x, batch):
    return _seg_sum(x, batch.astype(jnp.int32))


# speedup vs baseline: 1.2203x; 1.2203x over previous
import functools

import jax
import jax.numpy as jnp
from jax import lax
from jax.experimental import pallas as pl
from jax.experimental.pallas import tpu as pltpu
from jax.experimental.pallas import tpu_sc as plsc

NUM_ROWS = 100000
NUM_COLS = 128
NUM_SEG = 512
NC = 2
NS = 16
ROWS_PER_CORE = NUM_ROWS // NC
BLK = 208
FULL_BLKS = ROWS_PER_CORE // BLK
BLKS_PER_TILE = FULL_BLKS // NS
REM = ROWS_PER_CORE - FULL_BLKS * BLK
SEG_PER_TILE = NUM_SEG // NS
NBUF = 2
NCG = NUM_COLS // 16
GRP = 16


def _run_reduce_rows(xb, ib, nrows, carry, acc_l):
    iota16 = lax.iota(jnp.int32, 16)

    def row_step(r, carry):
        curb, accs = carry
        idb = plsc.load_gather(ib, [jnp.zeros((16,), jnp.int32) + r])
        fl = idb != curb
        for j in range(NCG):
            plsc.addupdate_scatter(
                acc_l, [curb, iota16 + 16 * j], accs[j], mask=fl)
        new_accs = []
        for j in range(NCG):
            xr = xb[r, pl.ds(j * 16, 16)]
            new_accs.append(jnp.where(fl, xr, accs[j] + xr))
        return idb, new_accs

    def group(u, carry):
        curb, accs = carry
        r0 = u * GRP
        idvec = ib[pl.ds(r0, GRP)]
        n_same = jnp.sum((idvec == curb).astype(jnp.int32))

        def fast(carry):
            curb, accs = carry
            for v in range(GRP):
                accs = [accs[j] + xb[r0 + v, pl.ds(j * 16, 16)]
                        for j in range(NCG)]
            return curb, accs

        def slow(carry):
            for v in range(GRP):
                carry = row_step(r0 + v, carry)
            return carry

        return lax.cond(n_same == GRP, fast, slow, (curb, accs))

    return lax.fori_loop(0, nrows // GRP, group, carry)


def _seg_sum_body(x_hbm, ids_hbm, part_hbm, scratch):
    (idxs, xbs, acc_l, acc_sh, ibuf, sld) = scratch
    c = lax.axis_index("c")
    s = lax.axis_index("s")

    base = c * ROWS_PER_CORE + s * (BLKS_PER_TILE * BLK)

    def start(b, k):
        row0 = pl.multiple_of(base + b * BLK, 8)
        pltpu.async_copy(ids_hbm.at[pl.ds(row0, BLK)], idxs[k], sld[k])
        pltpu.async_copy(x_hbm.at[pl.ds(row0, BLK)], xbs[k], sld[k])

    def wait(k):
        pltpu.make_async_copy(ids_hbm.at[pl.ds(0, BLK)], idxs[k], sld[k]).wait()
        pltpu.make_async_copy(x_hbm.at[pl.ds(0, BLK)], xbs[k], sld[k]).wait()

    for k in range(NBUF):
        start(k, k)

    def zrow_l(i, carry):
        for j in range(NCG):
            acc_l[i, pl.ds(j * 16, 16)] = jnp.zeros((16,), jnp.float32)
        return carry

    lax.fori_loop(0, NUM_SEG + 1, zrow_l, 0)

    pltpu.sync_copy(acc_l.at[pl.ds(s * SEG_PER_TILE, SEG_PER_TILE)],
                    acc_sh.at[pl.ds(s * SEG_PER_TILE, SEG_PER_TILE)])

    for q in range(NUM_SEG // 128):
        for t in range(128 // 16):
            ibuf[q, pl.ds(t * 16, 16)] = (
                lax.iota(jnp.int32, 16) + (q * 128 + t * 16))

    plsc.subcore_barrier()

    zero_accs = [jnp.zeros((16,), jnp.float32) for _ in range(NCG)]
    sentinel = jnp.full((16,), NUM_SEG, jnp.int32)
    carry = (sentinel, zero_accs)

    def grp_loop(g, carry):
        for k in range(NBUF):
            b = NBUF * g + k
            wait(k)
            carry = _run_reduce_rows(xbs[k], idxs[k], BLK, carry, acc_l)

            @pl.when(b + NBUF < BLKS_PER_TILE)
            def _():
                start(b + NBUF, k)
        return carry

    carry = lax.fori_loop(0, BLKS_PER_TILE // NBUF, grp_loop, carry)

    wait(0)
    carry = _run_reduce_rows(xbs[0], idxs[0], BLK, carry, acc_l)

    def rem_case(carry):
        row0 = pl.multiple_of(c * ROWS_PER_CORE + FULL_BLKS * BLK, 8)
        pltpu.sync_copy(ids_hbm.at[pl.ds(row0, REM)],
                        idxs[0].at[pl.ds(0, REM)])
        pltpu.sync_copy(x_hbm.at[pl.ds(row0, REM)], xbs[0].at[pl.ds(0, REM)])
        return _run_reduce_rows(xbs[0], idxs[0], REM, carry, acc_l)

    carry = lax.cond(s == NS - 1, rem_case, lambda car: car, carry)

    curb, accs = carry
    iota16 = lax.iota(jnp.int32, 16)
    for j in range(NCG):
        plsc.addupdate_scatter(acc_l, [curb, iota16 + 16 * j], accs[j])

    for q in range(NUM_SEG // 128):
        pltpu.sync_copy(acc_l.at[pl.ds(q * 128, 128)],
                        acc_sh.at[ibuf.at[q]], add=True)

    plsc.subcore_barrier()

    @pl.when(s == 0)
    def _():
        pltpu.sync_copy(acc_sh, part_hbm.at[c])


def _body_wrapper(x_hbm, ids_hbm, part_hbm,
                  i0, i1, x0, x1, acc_l, acc_sh, ibuf, l0, l1):
    _seg_sum_body(x_hbm, ids_hbm, part_hbm,
                  ((i0, i1), (x0, x1), acc_l, acc_sh, ibuf, (l0, l1)))


def _combine_body(p_ref, o_ref):
    o_ref[...] = p_ref[0] + p_ref[1]


@jax.jit
def _seg_sum(x, ids):
    mesh = plsc.VectorSubcoreMesh(core_axis_name="c", subcore_axis_name="s")
    parts = functools.partial(
        pl.kernel,
        out_type=jax.ShapeDtypeStruct((NC, NUM_SEG, NUM_COLS), jnp.float32),
        mesh=mesh,
        compiler_params=pltpu.CompilerParams(needs_layout_passes=False),
        scratch_types=(
            [pltpu.VMEM((BLK,), jnp.int32)] * 2
            + [pltpu.VMEM((BLK, NUM_COLS), jnp.float32)] * 2
            + [pltpu.VMEM((NUM_SEG + 1, NUM_COLS), jnp.float32)]
            + [pltpu.VMEM_SHARED((NUM_SEG, NUM_COLS), jnp.float32)]
            + [pltpu.VMEM((NUM_SEG // 128, 128), jnp.int32)]
            + [pltpu.SemaphoreType.DMA] * 2
        ),
    )(_body_wrapper)(x, ids)
    return pl.pallas_call(
        _combine_body,
        out_shape=jax.ShapeDtypeStruct((NUM_SEG, NUM_COLS), jnp.float32),
    )(parts)


def kernel(x, batch):
    return _seg_sum(x, batch.astype(jnp.int32))
